# baseline (speedup 1.0000x reference)
import functools

import jax
import jax.numpy as jnp
from jax import lax
from jax.experimental import pallas as pl
from jax.experimental.pallas import tpu as pltpu
from jax.experimental.pallas import tpu_sc as plsc

NV, D, NN, B = 2, 16, 1000000, 4096
NH, NR, NG = 5, 3, 10
NW = 32
PAIRS = 2 * B
PPW = PAIRS // NW
CH = 8
NCHUNK = PPW // CH
KN = 112
KE = 80
GN = KN // 16
GE = KE // 16
G = GN + GE

_mesh = plsc.VectorSubcoreMesh(
    core_axis_name="c", subcore_axis_name="s", num_cores=2, num_subcores=16)


def _splat(v):
    return jnp.full((16,), v, jnp.int32)


def _log_sigmoid(x):
    u = jnp.exp(-jnp.abs(x))
    t = 1.0 + u
    bits = lax.bitcast_convert_type(t, jnp.int32)
    ex = ((bits >> 23) - 127).astype(jnp.float32)
    m = lax.bitcast_convert_type((bits & 0x007FFFFF) | 0x3F800000, jnp.float32)
    s = (m - 1.0) / (m + 1.0)
    s2 = s * s
    p = jnp.float32(1.0 / 9.0)
    p = jnp.float32(1.0 / 7.0) + s2 * p
    p = jnp.float32(1.0 / 5.0) + s2 * p
    p = jnp.float32(1.0 / 3.0) + s2 * p
    p = jnp.float32(1.0) + s2 * p
    l1p = ex * jnp.float32(0.6931471805599453) + 2.0 * s * p
    return jnp.minimum(x, 0.0) - l1p


@functools.partial(
    pl.kernel,
    out_type=jax.ShapeDtypeStruct((NW, 16), jnp.float32),
    mesh=_mesh,
    compiler_params=pltpu.CompilerParams(
        use_tc_tiling_on_sc=False, needs_layout_passes=False),
    scratch_types=[
        pltpu.VMEM((2, 128), jnp.int32),
        pltpu.VMEM((PPW, 16), jnp.float32),
        pltpu.VMEM((GN, 128), jnp.int32),
        pltpu.VMEM((GE, 128), jnp.int32),
        pltpu.VMEM((CH * KN, 16), jnp.float32),
        pltpu.VMEM((CH * KE, 16), jnp.float32),
        pltpu.VMEM((2 * G, 16), jnp.float32),
        pltpu.VMEM((16,), jnp.float32),
        pltpu.SemaphoreType.DMA,
    ],
)
def _sc_loss(nodes_hbm, neigh_hbm, qidx_hbm, nidx_hbm, eidx_hbm, ws_hbm, out_hbm,
             qidx_v, qbuf, nidx_v, eidx_v, nrows, erows, wsbuf, accbuf, sem):
    wid = lax.axis_index("s") * 2 + lax.axis_index("c")
    pltpu.sync_copy(ws_hbm, wsbuf)
    pltpu.sync_copy(qidx_hbm.at[wid], qidx_v)
    for k in range(2):
        pltpu.async_copy(nodes_hbm.at[qidx_v.at[k]],
                         qbuf.at[pl.ds(k * 128, 128)], sem).wait()
    iota16 = lax.iota(jnp.int32, 16)

    def chunk_body(c, acc):
        pltpu.sync_copy(nidx_hbm.at[wid, c], nidx_v)
        pltpu.sync_copy(eidx_hbm.at[wid, c], eidx_v)
        copies = []
        for j in range(GN):
            copies.append(pltpu.async_copy(
                neigh_hbm.at[nidx_v.at[j]], nrows.at[pl.ds(j * 128, 128)], sem))
        for j in range(GE):
            copies.append(pltpu.async_copy(
                nodes_hbm.at[eidx_v.at[j]], erows.at[pl.ds(j * 128, 128)], sem))
        for cp in copies:
            cp.wait()

        def pair_body(p, acc2):
            prow = c * CH + p
            qd = [plsc.load_gather(qbuf, [_splat(prow), _splat(d)])
                  for d in range(16)]
            for g in range(G):
                if g < GN:
                    ridx = _splat(p * KN + g * 16) + iota16
                    rref = nrows
                else:
                    ridx = _splat(p * KE + (g - GN) * 16) + iota16
                    rref = erows
                dot = plsc.load_gather(rref, [ridx, _splat(0)]) * qd[0]
                for d in range(1, 16):
                    dot = dot + plsc.load_gather(rref, [ridx, _splat(d)]) * qd[d]
                vals = _log_sigmoid(dot * wsbuf[G + g])
                acc2 = acc2 + wsbuf[g] * vals
            return acc2

        return lax.fori_loop(0, CH, pair_body, acc)

    acc = lax.fori_loop(0, NCHUNK, chunk_body, jnp.zeros((16,), jnp.float32))
    accbuf[...] = acc
    pltpu.sync_copy(accbuf, out_hbm.at[wid])


def kernel(node_emb_tables, neigh_emb_tables, hyp1, hyp2, hyp3, count,
           shuffle_indices_nets, nodes_idx_nets, neigh_idx_nets,
           node_role_nets, neg_main, neg2, neg3, neg4):
    nodes_flat = node_emb_tables.reshape(NV * NN, D)
    neigh_flat = neigh_emb_tables.reshape(NV * NN, D)

    qidx_l, nei_l, nod_l = [], [], []
    for i in range(NV):
        j = 1 - i
        bidx = lax.dynamic_slice_in_dim(shuffle_indices_nets[i], count, B)
        nodes_idx = nodes_idx_nets[i][bidx]
        neighs_idx = neigh_idx_nets[i][bidx]
        role0 = node_role_nets[i, 0][bidx]
        role1 = node_role_nets[i, 1][bidx]
        qidx_l.append(i * NN + nodes_idx)
        nei_l.append(jnp.concatenate([
            i * NN + neighs_idx,
            i * NN + neg_main[i].reshape(B, NH * NG),
            j * NN + neighs_idx,
            j * NN + neg3[i, j].reshape(B, NH * NG),
            jnp.zeros((B, 2), jnp.int32)], axis=1))
        nod_l.append(jnp.concatenate([
            (j * NN + nodes_idx)[:, None],
            j * NN + neg2[i, j].reshape(B, NG),
            role0,
            NN + role1,
            neg4[i, 0].reshape(B, NR * NG),
            NN + neg4[i, 1].reshape(B, NR * NG),
            jnp.zeros((B, 3), jnp.int32)], axis=1))
    qidx = jnp.concatenate(qidx_l).astype(jnp.int32).reshape(NW, 2, 128)
    nei = jnp.concatenate(nei_l).astype(jnp.int32).reshape(NW, NCHUNK, GN, 128)
    nod = jnp.concatenate(nod_l).astype(jnp.int32).reshape(NW, NCHUNK, GE, 128)

    Bf = jnp.float32(B)
    one = jnp.float32(1.0)
    w = jnp.concatenate([
        jnp.full(5, one / (NH * Bf)), jnp.full(50, one / Bf),
        jnp.full(5, hyp2 / (NH * Bf)), jnp.full(50, hyp2 / Bf), jnp.zeros(2),
        jnp.full(1, hyp1 / Bf), jnp.full(10, hyp1 / Bf),
        jnp.full(6, hyp3 / (NR * Bf)), jnp.full(60, hyp3 / Bf), jnp.zeros(3),
    ]).astype(jnp.float32) * jnp.float32(-0.1)
    s = jnp.concatenate([
        jnp.full(5, 1.0), jnp.full(50, -1.0),
        jnp.full(5, 1.0), jnp.full(50, -1.0), jnp.full(2, 1.0),
        jnp.full(1, 1.0), jnp.full(10, -1.0),
        jnp.full(6, 1.0), jnp.full(60, -1.0), jnp.full(3, 1.0),
    ]).astype(jnp.float32)
    ws = jnp.concatenate([w.reshape(G, 16), s.reshape(G, 16)], axis=0)

    out = _sc_loss(nodes_flat, neigh_flat, qidx, nei, nod, ws)
    return jnp.sum(out)


# speedup vs baseline: 1.0292x; 1.0292x over previous
import functools

import jax
import jax.numpy as jnp
from jax import lax
from jax.experimental import pallas as pl
from jax.experimental.pallas import tpu as pltpu
from jax.experimental.pallas import tpu_sc as plsc

NV, D, NN, B = 2, 16, 1000000, 4096
NH, NR, NG = 5, 3, 10
NW = 32
WPV = NW // NV
PPW = B // WPV
CE = 256
_REG_NODES, _REG_NEIGH, _REG_ROLE0, _REG_ROLE1 = 0, 1, 6, 9
_REG_NEGM, _REG_NEG2, _REG_NEG3, _REG_NEG4A, _REG_NEG4B = 12, 62, 72, 122, 152
N_CHUNKS = 182
IDXW = N_CHUNKS * CE

_mesh = plsc.VectorSubcoreMesh(
    core_axis_name="c", subcore_axis_name="s", num_cores=2, num_subcores=16)


def _splat(v):
    return jnp.full((16,), v, jnp.int32)


def _log_sigmoid(x):
    u = jnp.exp(-jnp.abs(x))
    t = 1.0 + u
    bits = lax.bitcast_convert_type(t, jnp.int32)
    ex = ((bits >> 23) - 127).astype(jnp.float32)
    m = lax.bitcast_convert_type((bits & 0x007FFFFF) | 0x3F800000, jnp.float32)
    s = (m - 1.0) / (m + 1.0)
    s2 = s * s
    p = jnp.float32(1.0 / 9.0)
    p = jnp.float32(1.0 / 7.0) + s2 * p
    p = jnp.float32(1.0 / 5.0) + s2 * p
    p = jnp.float32(1.0 / 3.0) + s2 * p
    p = jnp.float32(1.0) + s2 * p
    l1p = ex * jnp.float32(0.6931471805599453) + 2.0 * s * p
    return jnp.minimum(x, 0.0) - l1p


@functools.partial(
    pl.kernel,
    out_type=jax.ShapeDtypeStruct((NW, 16), jnp.float32),
    mesh=_mesh,
    compiler_params=pltpu.CompilerParams(
        use_tc_tiling_on_sc=False, needs_layout_passes=False),
    scratch_types=[
        pltpu.VMEM((IDXW,), jnp.int32),
        pltpu.VMEM((PPW, 16), jnp.float32),
        pltpu.VMEM((2, CE, 16), jnp.float32),
        pltpu.VMEM((16,), jnp.float32),
        pltpu.VMEM((16,), jnp.float32),
        pltpu.SemaphoreType.DMA,
        pltpu.SemaphoreType.DMA,
        pltpu.SemaphoreType.DMA((2,)),
    ],
)
def _sc_loss(node_t, neigh_t, nodes_ib, neighs_ib, roles_ib,
             negm, neg2, neg3, neg4, wvec_hbm, out_hbm,
             idxflat, qbuf, rows, wbuf, accv, sems, semq, semr):
    wid = lax.axis_index("s") * 2 + lax.axis_index("c")
    pltpu.sync_copy(wvec_hbm, wbuf)
    iota16 = lax.iota(jnp.int32, 16)

    def compute_chunk(c, par, seg_chunk0, inv_k, sign, wsplat, acc):
        psplat = _splat(par)
        ebase = (c - seg_chunk0) * CE

        def gbody(g, acc2):
            ridx = _splat(g * 16) + iota16
            e_f = (_splat(ebase + g * 16) + iota16).astype(jnp.float32)
            bvec = ((e_f + 0.5) * inv_k).astype(jnp.int32)
            dot = (plsc.load_gather(rows, [psplat, ridx, _splat(0)])
                   * plsc.load_gather(qbuf, [bvec, _splat(0)]))
            for d in range(1, 16):
                dot = dot + (plsc.load_gather(rows, [psplat, ridx, _splat(d)])
                             * plsc.load_gather(qbuf, [bvec, _splat(d)]))
            x = dot if sign > 0 else -dot
            return acc2 + wsplat * _log_sigmoid(x)

        return lax.fori_loop(0, 16, gbody, acc)

    def run_segment(tab, chunk0, n, inv_k, sign, wslot, acc):
        wsplat = plsc.load_gather(wbuf, [_splat(wslot)])

        def issue(c, par):
            for q in range(2):
                pltpu.async_copy(
                    tab.at[idxflat.at[pl.ds((chunk0 + c) * CE + q * 128, 128)]],
                    rows.at[par, pl.ds(q * 128, 128)], semr.at[par])

        def drain(par):
            for q in range(2):
                pltpu.make_async_copy(
                    node_t.at[0, pl.ds(0, 128)],
                    rows.at[par, pl.ds(q * 128, 128)], semr.at[par]).wait()

        issue(0, 0)

        def cbody(c, acc2):
            par = lax.rem(c, 2)

            @pl.when(c + 1 < n)
            def _():
                issue(c + 1, lax.rem(c + 1, 2))

            drain(par)
            return compute_chunk(chunk0 + c, par, chunk0, inv_k, sign,
                                 wsplat, acc2)

        return lax.fori_loop(0, n, cbody, acc)

    def block(i):
        j = 1 - i
        wl = wid - i * WPV
        b0 = wl * PPW
        regions = [
            (_REG_NODES, 1, nodes_ib.at[i, pl.ds(b0, PPW)]),
            (_REG_NEIGH, 5, neighs_ib.at[i, pl.ds(b0 * NH, PPW * NH)]),
            (_REG_ROLE0, 3, roles_ib.at[i, 0, pl.ds(b0 * NR, PPW * NR)]),
            (_REG_ROLE1, 3, roles_ib.at[i, 1, pl.ds(b0 * NR, PPW * NR)]),
            (_REG_NEGM, 50, negm.at[i, pl.ds(b0 * NH * NG, PPW * NH * NG)]),
            (_REG_NEG2, 10, neg2.at[i, j, pl.ds(b0 * NG, PPW * NG)]),
            (_REG_NEG3, 50, neg3.at[i, j, pl.ds(b0 * NH * NG, PPW * NH * NG)]),
            (_REG_NEG4A, 30, neg4.at[i, 0, pl.ds(b0 * NR * NG, PPW * NR * NG)]),
            (_REG_NEG4B, 30, neg4.at[i, 1, pl.ds(b0 * NR * NG, PPW * NR * NG)]),
        ]
        for base, n, src in regions:
            pltpu.async_copy(src, idxflat.at[pl.ds(base * CE, n * CE)], sems)
        for base, n, src in regions:
            pltpu.make_async_copy(
                negm.at[0, pl.ds(0, n * CE)],
                idxflat.at[pl.ds(base * CE, n * CE)], sems).wait()
        for q in range(2):
            pltpu.async_copy(
                node_t.at[i].at[idxflat.at[pl.ds(_REG_NODES * CE + q * 128, 128)]],
                qbuf.at[pl.ds(q * 128, 128)], semq)
        for q in range(2):
            pltpu.make_async_copy(node_t.at[0, pl.ds(0, 128)],
                                  qbuf.at[pl.ds(q * 128, 128)], semq).wait()

        inv5 = jnp.float32(1.0 / NH)
        inv50 = jnp.float32(1.0 / (NH * NG))
        inv10 = jnp.float32(1.0 / NG)
        inv3 = jnp.float32(1.0 / NR)
        inv30 = jnp.float32(1.0 / (NR * NG))
        one = jnp.float32(1.0)
        segs = [
            (neigh_t.at[i], _REG_NEIGH, 5, inv5, 1, 0),
            (neigh_t.at[i], _REG_NEGM, 50, inv50, -1, 1),
            (node_t.at[j], _REG_NODES, 1, one, 1, 2),
            (node_t.at[j], _REG_NEG2, 10, inv10, -1, 2),
            (neigh_t.at[j], _REG_NEIGH, 5, inv5, 1, 3),
            (neigh_t.at[j], _REG_NEG3, 50, inv50, -1, 4),
            (node_t.at[0], _REG_ROLE0, 3, inv3, 1, 5),
            (node_t.at[1], _REG_ROLE1, 3, inv3, 1, 5),
            (node_t.at[0], _REG_NEG4A, 30, inv30, -1, 6),
            (node_t.at[1], _REG_NEG4B, 30, inv30, -1, 6),
        ]
        acc = jnp.zeros((16,), jnp.float32)
        for tab, chunk0, n, inv_k, sign, wslot in segs:
            acc = run_segment(tab, chunk0, n, inv_k, sign, wslot, acc)
        accv[...] = acc

    @pl.when(wid < WPV)
    def _():
        block(0)

    @pl.when(wid >= WPV)
    def _():
        block(1)

    pltpu.sync_copy(accv, out_hbm.at[wid])


def kernel(node_emb_tables, neigh_emb_tables, hyp1, hyp2, hyp3, count,
           shuffle_indices_nets, nodes_idx_nets, neigh_idx_nets,
           node_role_nets, neg_main, neg2, neg3, neg4):
    bidx = [lax.dynamic_slice_in_dim(shuffle_indices_nets[i], count, B)
            for i in range(NV)]
    nodes_ib = jnp.stack([nodes_idx_nets[i][bidx[i]] for i in range(NV)])
    neighs_ib = jnp.stack(
        [neigh_idx_nets[i][bidx[i]].reshape(-1) for i in range(NV)])
    roles_ib = jnp.stack([
        jnp.stack([node_role_nets[i, jj][bidx[i]].reshape(-1)
                   for jj in range(NV)]) for i in range(NV)])

    Bf = jnp.float32(B)
    scale = jnp.float32(-0.1)
    wvec = jnp.stack([
        1.0 / (NH * Bf), 1.0 / Bf, hyp1 / Bf, hyp2 / (NH * Bf), hyp2 / Bf,
        hyp3 / (NR * Bf), hyp3 / Bf,
        0.0, 0.0, 0.0, 0.0, 0.0, 0.0, 0.0, 0.0, 0.0,
    ]).astype(jnp.float32) * scale

    out = _sc_loss(node_emb_tables, neigh_emb_tables,
                   nodes_ib.astype(jnp.int32), neighs_ib.astype(jnp.int32),
                   roles_ib.astype(jnp.int32), neg_main, neg2, neg3, neg4,
                   wvec)
    return jnp.sum(out)


# speedup vs baseline: 1.0354x; 1.0061x over previous
import functools

import jax
import jax.numpy as jnp
from jax import lax
from jax.experimental import pallas as pl
from jax.experimental.pallas import tpu as pltpu
from jax.experimental.pallas import tpu_sc as plsc

NV, D, NN, B = 2, 16, 1000000, 4096
NH, NR, NG = 5, 3, 10
NW = 32
WPV = NW // NV
PPW = B // WPV
CE = 256
_REG_NODES, _REG_NEIGH, _REG_ROLE0, _REG_ROLE1 = 0, 1, 6, 9
_REG_NEGM, _REG_NEG2, _REG_NEG3, _REG_NEG4A, _REG_NEG4B = 12, 62, 72, 122, 152
N_CHUNKS = 182
IDXW = N_CHUNKS * CE

_mesh = plsc.VectorSubcoreMesh(
    core_axis_name="c", subcore_axis_name="s", num_cores=2, num_subcores=16)


def _splat(v):
    return jnp.full((16,), v, jnp.int32)


def _log_sigmoid(x):
    u = jnp.exp(-jnp.abs(x))
    t = 1.0 + u
    bits = lax.bitcast_convert_type(t, jnp.int32)
    ex = ((bits >> 23) - 127).astype(jnp.float32)
    m = lax.bitcast_convert_type((bits & 0x007FFFFF) | 0x3F800000, jnp.float32)
    s = (m - 1.0) / (m + 1.0)
    s2 = s * s
    p = jnp.float32(1.0 / 9.0)
    p = jnp.float32(1.0 / 7.0) + s2 * p
    p = jnp.float32(1.0 / 5.0) + s2 * p
    p = jnp.float32(1.0 / 3.0) + s2 * p
    p = jnp.float32(1.0) + s2 * p
    l1p = ex * jnp.float32(0.6931471805599453) + 2.0 * s * p
    return jnp.minimum(x, 0.0) - l1p


@functools.partial(
    pl.kernel,
    out_type=jax.ShapeDtypeStruct((NW, 16), jnp.float32),
    mesh=_mesh,
    compiler_params=pltpu.CompilerParams(
        use_tc_tiling_on_sc=False, needs_layout_passes=False),
    scratch_types=[
        pltpu.VMEM((IDXW,), jnp.int32),
        pltpu.VMEM((PPW, 16), jnp.float32),
        pltpu.VMEM((2, 2 * CE, 16), jnp.float32),
        pltpu.VMEM((16,), jnp.float32),
        pltpu.VMEM((16,), jnp.float32),
        pltpu.SemaphoreType.DMA,
        pltpu.SemaphoreType.DMA,
        pltpu.SemaphoreType.DMA((2,)),
    ],
)
def _sc_loss(node_t, neigh_t, nodes_ib, neighs_ib, roles_ib,
             negm, neg2, neg3, neg4, wvec_hbm, out_hbm,
             idxflat, qbuf, rows, wbuf, accv, sems, semq, semr):
    wid = lax.axis_index("s") * 2 + lax.axis_index("c")
    pltpu.sync_copy(wvec_hbm, wbuf)
    iota16 = lax.iota(jnp.int32, 16)

    def compute_chunk(c, par, ce, inv_k, sign, wsplat, acc):
        psplat = _splat(par)
        ebase0 = c * ce

        def gbody(g, acc2):
            ridx = _splat(g * 16) + iota16
            e_f = (_splat(g * 16) + iota16).astype(jnp.float32) + (
                ebase0.astype(jnp.float32))
            bvec = ((e_f + 0.5) * inv_k).astype(jnp.int32)
            dot = (plsc.load_gather(rows, [psplat, ridx, _splat(0)])
                   * plsc.load_gather(qbuf, [bvec, _splat(0)]))
            for d in range(1, 16):
                dot = dot + (plsc.load_gather(rows, [psplat, ridx, _splat(d)])
                             * plsc.load_gather(qbuf, [bvec, _splat(d)]))
            x = dot if sign > 0 else -dot
            return acc2 + wsplat * _log_sigmoid(x)

        return lax.fori_loop(0, ce // 16, gbody, acc)

    def run_segment(tab, ebase, n, ce, inv_k, sign, wslot, acc):
        wsplat = plsc.load_gather(wbuf, [_splat(wslot)])

        def issue(c, par):
            pltpu.async_copy(
                tab.at[idxflat.at[pl.ds(ebase + c * ce, ce)]],
                rows.at[par, pl.ds(0, ce)], semr.at[par])

        def drain(par):
            pltpu.make_async_copy(
                node_t.at[0, pl.ds(0, ce)],
                rows.at[par, pl.ds(0, ce)], semr.at[par]).wait()

        issue(0, 0)

        def cbody(c, acc2):
            par = lax.rem(c, 2)

            @pl.when(c + 1 < n)
            def _():
                issue(c + 1, lax.rem(c + 1, 2))

            drain(par)
            return compute_chunk(c, par, ce, inv_k, sign, wsplat, acc2)

        return lax.fori_loop(0, n, cbody, acc)

    def block(i):
        j = 1 - i
        wl = wid - i * WPV
        b0 = wl * PPW
        regions = [
            (_REG_NODES, 1, nodes_ib.at[i, pl.ds(b0, PPW)]),
            (_REG_NEIGH, 5, neighs_ib.at[i, pl.ds(b0 * NH, PPW * NH)]),
            (_REG_ROLE0, 3, roles_ib.at[i, 0, pl.ds(b0 * NR, PPW * NR)]),
            (_REG_ROLE1, 3, roles_ib.at[i, 1, pl.ds(b0 * NR, PPW * NR)]),
            (_REG_NEGM, 50, negm.at[i, pl.ds(b0 * NH * NG, PPW * NH * NG)]),
            (_REG_NEG2, 10, neg2.at[i, j, pl.ds(b0 * NG, PPW * NG)]),
            (_REG_NEG3, 50, neg3.at[i, j, pl.ds(b0 * NH * NG, PPW * NH * NG)]),
            (_REG_NEG4A, 30, neg4.at[i, 0, pl.ds(b0 * NR * NG, PPW * NR * NG)]),
            (_REG_NEG4B, 30, neg4.at[i, 1, pl.ds(b0 * NR * NG, PPW * NR * NG)]),
        ]
        for base, n, src in regions:
            pltpu.async_copy(src, idxflat.at[pl.ds(base * CE, n * CE)], sems)
        for base, n, src in regions:
            pltpu.make_async_copy(
                negm.at[0, pl.ds(0, n * CE)],
                idxflat.at[pl.ds(base * CE, n * CE)], sems).wait()
        for q in range(2):
            pltpu.async_copy(
                node_t.at[i].at[idxflat.at[pl.ds(_REG_NODES * CE + q * 128, 128)]],
                qbuf.at[pl.ds(q * 128, 128)], semq)
        for q in range(2):
            pltpu.make_async_copy(node_t.at[0, pl.ds(0, 128)],
                                  qbuf.at[pl.ds(q * 128, 128)], semq).wait()

        inv5 = jnp.float32(1.0 / NH)
        inv50 = jnp.float32(1.0 / (NH * NG))
        inv10 = jnp.float32(1.0 / NG)
        inv3 = jnp.float32(1.0 / NR)
        inv30 = jnp.float32(1.0 / (NR * NG))
        one = jnp.float32(1.0)
        segs = [
            (neigh_t.at[i], _REG_NEIGH, 5, 256, inv5, 1, 0),
            (neigh_t.at[i], _REG_NEGM, 25, 512, inv50, -1, 1),
            (node_t.at[j], _REG_NODES, 1, 256, one, 1, 2),
            (node_t.at[j], _REG_NEG2, 5, 512, inv10, -1, 2),
            (neigh_t.at[j], _REG_NEIGH, 5, 256, inv5, 1, 3),
            (neigh_t.at[j], _REG_NEG3, 25, 512, inv50, -1, 4),
            (node_t.at[0], _REG_ROLE0, 3, 256, inv3, 1, 5),
            (node_t.at[1], _REG_ROLE1, 3, 256, inv3, 1, 5),
            (node_t.at[0], _REG_NEG4A, 15, 512, inv30, -1, 6),
            (node_t.at[1], _REG_NEG4B, 15, 512, inv30, -1, 6),
        ]
        acc = jnp.zeros((16,), jnp.float32)
        for tab, reg0, n, ce, inv_k, sign, wslot in segs:
            acc = run_segment(tab, reg0 * CE, n, ce, inv_k, sign, wslot, acc)
        accv[...] = acc

    @pl.when(wid < WPV)
    def _():
        block(0)

    @pl.when(wid >= WPV)
    def _():
        block(1)

    pltpu.sync_copy(accv, out_hbm.at[wid])


def kernel(node_emb_tables, neigh_emb_tables, hyp1, hyp2, hyp3, count,
           shuffle_indices_nets, nodes_idx_nets, neigh_idx_nets,
           node_role_nets, neg_main, neg2, neg3, neg4):
    bidx = [lax.dynamic_slice_in_dim(shuffle_indices_nets[i], count, B)
            for i in range(NV)]
    nodes_ib = jnp.stack([nodes_idx_nets[i][bidx[i]] for i in range(NV)])
    neighs_ib = jnp.stack(
        [neigh_idx_nets[i][bidx[i]].reshape(-1) for i in range(NV)])
    roles_ib = jnp.stack([
        jnp.stack([node_role_nets[i, jj][bidx[i]].reshape(-1)
                   for jj in range(NV)]) for i in range(NV)])

    Bf = jnp.float32(B)
    scale = jnp.float32(-0.1)
    wvec = jnp.stack([
        1.0 / (NH * Bf), 1.0 / Bf, hyp1 / Bf, hyp2 / (NH * Bf), hyp2 / Bf,
        hyp3 / (NR * Bf), hyp3 / Bf,
        0.0, 0.0, 0.0, 0.0, 0.0, 0.0, 0.0, 0.0, 0.0,
    ]).astype(jnp.float32) * scale

    out = _sc_loss(node_emb_tables, neigh_emb_tables,
                   nodes_ib.astype(jnp.int32), neighs_ib.astype(jnp.int32),
                   roles_ib.astype(jnp.int32), neg_main, neg2, neg3, neg4,
                   wvec)
    return jnp.sum(out)
